# baseline (speedup 1.0000x reference)
import jax
import jax.numpy as jnp
from jax import lax
from jax.experimental import pallas as pl
from jax.experimental.pallas import tpu as pltpu
from jax.experimental.pallas import tpu_sc as plsc

E = 8
K = 2
D = 768
BS = 128
DFFN = 1536
T = 2048
NPAIR = T * K
NB = 40
P = NB * BS
TBLK = 128
NTB = T // TBLK
NW = 32


def _router_body(x_ref, rwt_ref, sel_ref, rank_ref, rw_ref, cnt0_ref,
                 offs_ref, be_ref, act_ref, carry0, carry1):
    i = pl.program_id(0)

    @pl.when(i == 0)
    def _():
        carry0[...] = jnp.zeros_like(carry0)
        carry1[...] = jnp.zeros_like(carry1)

    xb = x_ref[...]
    logits = jnp.dot(xb, rwt_ref[...], preferred_element_type=jnp.float32)
    m = jnp.max(logits, axis=1, keepdims=True)
    ex = jnp.exp(logits - m)
    p = ex / jnp.sum(ex, axis=1, keepdims=True)
    lane8 = lax.broadcasted_iota(jnp.int32, (TBLK, E), 1)
    m1 = jnp.max(p, axis=1, keepdims=True)
    i1 = jnp.min(jnp.where(p >= m1, lane8, E), axis=1, keepdims=True)
    p2 = jnp.where(lane8 == i1, -1.0, p)
    m2 = jnp.max(p2, axis=1, keepdims=True)
    i2 = jnp.min(jnp.where(p2 >= m2, lane8, E), axis=1, keepdims=True)
    ssum = m1 + m2
    sel_ref[0] = i1
    sel_ref[1] = i2
    rw_ref[0] = m1 / ssum
    rw_ref[1] = m2 / ssum

    lane128 = lax.broadcasted_iota(jnp.int32, (TBLK, 128), 1)
    row128 = lax.broadcasted_iota(jnp.int32, (TBLK, 128), 0)
    tril = (row128 >= lane128).astype(jnp.float32)
    for g, (sel, carry) in enumerate(((i1, carry0), (i2, carry1))):
        oh = (sel == lane128).astype(jnp.float32)
        cum = jnp.dot(tril, oh, preferred_element_type=jnp.float32)
        cb = carry[...]
        rank = jnp.sum(oh * (cum + cb - 1.0), axis=1, keepdims=True)
        rank_ref[g] = rank.astype(jnp.int32)
        carry[...] = cb + jnp.sum(oh, axis=0, keepdims=True)

    @pl.when(i == NTB - 1)
    def _():
        c0 = carry0[...]
        tot = (c0 + carry1[...]).astype(jnp.int32)
        padded = ((tot + 127) >> 7) << 7
        nblk = (padded >> 7).astype(jnp.float32)
        mstrict = (row128 < lane128).astype(jnp.float32)
        mincl = (row128 <= lane128).astype(jnp.float32)
        offs = jnp.dot(padded.astype(jnp.float32), mstrict,
                       preferred_element_type=jnp.float32)
        cnt0_ref[...] = c0.astype(jnp.int32).reshape(1, 1, 128)
        offs_ref[...] = offs.astype(jnp.int32).reshape(1, 1, 128)
        bo = jnp.dot(nblk, mincl, preferred_element_type=jnp.float32)
        lm = lane128 < E
        cmp = jnp.logical_and(row128.astype(jnp.float32) >= bo, lm)
        be = jnp.sum(cmp.astype(jnp.float32), axis=1, keepdims=True)
        be_ref[...] = jnp.minimum(be, float(E - 1)).astype(jnp.int32)
        nbtot = jnp.sum(jnp.where(lm[0:1, :], nblk, 0.0), axis=1,
                        keepdims=True)
        act_ref[...] = (row128[:, 0:1].astype(jnp.float32) < nbtot
                        ).astype(jnp.int32)


def _router(x2d, rwt):
    return pl.pallas_call(
        _router_body,
        grid=(NTB,),
        in_specs=[
            pl.BlockSpec((TBLK, D), lambda i: (i, 0)),
            pl.BlockSpec((D, E), lambda i: (0, 0)),
        ],
        out_specs=[
            pl.BlockSpec((K, TBLK, 1), lambda i: (0, i, 0)),
            pl.BlockSpec((K, TBLK, 1), lambda i: (0, i, 0)),
            pl.BlockSpec((K, TBLK, 1), lambda i: (0, i, 0)),
            pl.BlockSpec((1, 1, 128), lambda i: (0, 0, 0)),
            pl.BlockSpec((1, 1, 128), lambda i: (0, 0, 0)),
            pl.BlockSpec((128, 1), lambda i: (0, 0)),
            pl.BlockSpec((128, 1), lambda i: (0, 0)),
        ],
        out_shape=[
            jax.ShapeDtypeStruct((K, T, 1), jnp.int32),
            jax.ShapeDtypeStruct((K, T, 1), jnp.int32),
            jax.ShapeDtypeStruct((K, T, 1), jnp.float32),
            jax.ShapeDtypeStruct((1, 1, 128), jnp.int32),
            jax.ShapeDtypeStruct((1, 1, 128), jnp.int32),
            jax.ShapeDtypeStruct((128, 1), jnp.int32),
            jax.ShapeDtypeStruct((128, 1), jnp.int32),
        ],
        scratch_shapes=[
            pltpu.VMEM((1, 128), jnp.float32),
            pltpu.VMEM((1, 128), jnp.float32),
        ],
    )(x2d, rwt)


def _dispatch_body(sel_h, rank_h, cnt0_h, offs_h, x_h, xs_h, dst_h,
                   selc_v, rankc_v, c0_v, offs_v, dst_v, tok_v,
                   rows_v, sem1, sem2):
    wid = lax.axis_index("s") * 2 + lax.axis_index("c")
    kflag = wid // 16
    tb = (wid % 16) * 128
    pb = wid * 128
    pltpu.sync_copy(sel_h.at[pl.ds(pb, 128)], selc_v)
    pltpu.sync_copy(rank_h.at[pl.ds(pb, 128)], rankc_v)
    pltpu.sync_copy(cnt0_h.at[pl.ds(0, 16)], c0_v)
    pltpu.sync_copy(offs_h.at[pl.ds(0, 16)], offs_v)
    kvec = jnp.full((16,), kflag, dtype=jnp.int32)
    for j in range(8):
        s16 = selc_v[pl.ds(j * 16, 16)]
        r16 = rankc_v[pl.ds(j * 16, 16)]
        o16 = plsc.load_gather(offs_v, [s16])
        c016 = plsc.load_gather(c0_v, [s16])
        d16 = o16 + c016 * kvec + r16
        dst_v[pl.ds(j * 16, 16)] = d16
        tok_v[pl.ds(j * 16, 16)] = tb + j * 16 + lax.iota(jnp.int32, 16)
    pltpu.async_copy(x_h.at[tok_v], rows_v, sem1).wait()
    pltpu.async_copy(rows_v, xs_h.at[dst_v], sem2).wait()
    pltpu.sync_copy(dst_v, dst_h.at[pl.ds(pb, 128)])


def _dispatch(sel_flat, rank_flat, cnt0_last, offs_last, x2d):
    f = pl.kernel(
        _dispatch_body,
        out_type=[
            jax.ShapeDtypeStruct((P, D), jnp.float32),
            jax.ShapeDtypeStruct((NPAIR,), jnp.int32),
        ],
        mesh=plsc.VectorSubcoreMesh(core_axis_name="c", subcore_axis_name="s"),
        compiler_params=pltpu.CompilerParams(needs_layout_passes=False),
        scratch_types=[
            pltpu.VMEM((128,), jnp.int32),
            pltpu.VMEM((128,), jnp.int32),
            pltpu.VMEM((16,), jnp.int32),
            pltpu.VMEM((16,), jnp.int32),
            pltpu.VMEM((128,), jnp.int32),
            pltpu.VMEM((128,), jnp.int32),
            pltpu.VMEM((128, D), jnp.float32),
            pltpu.SemaphoreType.DMA,
            pltpu.SemaphoreType.DMA,
        ],
    )
    return f(sel_flat, rank_flat, cnt0_last, offs_last, x2d)


def _ffn_body(be_ref, act_ref, xs_ref, w1_ref, w2_ref, y_ref):
    b = pl.program_id(0)

    @pl.when(act_ref[b] == 1)
    def _():
        h = jnp.dot(xs_ref[...].astype(jnp.bfloat16),
                    w1_ref[...].astype(jnp.bfloat16),
                    preferred_element_type=jnp.float32)
        h = jax.nn.gelu(h)
        y_ref[...] = jnp.dot(h.astype(jnp.bfloat16),
                             w2_ref[...].astype(jnp.bfloat16),
                             preferred_element_type=jnp.float32)


def _ffn(be, act, xs, w1, w2):
    grid_spec = pltpu.PrefetchScalarGridSpec(
        num_scalar_prefetch=2,
        grid=(NB,),
        in_specs=[
            pl.BlockSpec((BS, D), lambda b, be_r, act_r: (b, 0)),
            pl.BlockSpec((D, DFFN), lambda b, be_r, act_r: (0, be_r[b])),
            pl.BlockSpec((DFFN, D), lambda b, be_r, act_r: (be_r[b], 0)),
        ],
        out_specs=pl.BlockSpec((BS, D), lambda b, be_r, act_r: (b, 0)),
    )
    return pl.pallas_call(
        _ffn_body,
        grid_spec=grid_spec,
        out_shape=jax.ShapeDtypeStruct((P, D), jnp.float32),
    )(be, act, xs, w1, w2)


def _combine_body(y_h, dst_h, rw_h, o_h, i0_v, i1_v, w0_v, w1_v,
                  r0_v, r1_v, sem1, sem2):
    wid = lax.axis_index("s") * 2 + lax.axis_index("c")
    tb = wid * 64
    pltpu.sync_copy(dst_h.at[pl.ds(tb, 64)], i0_v)
    pltpu.sync_copy(dst_h.at[pl.ds(T + tb, 64)], i1_v)
    pltpu.sync_copy(rw_h.at[pl.ds(tb, 64)], w0_v)
    pltpu.sync_copy(rw_h.at[pl.ds(T + tb, 64)], w1_v)
    pltpu.async_copy(y_h.at[i0_v], r0_v, sem1).wait()
    pltpu.async_copy(y_h.at[i1_v], r1_v, sem2).wait()

    def body(j, carry):
        jv = jnp.full((16,), j, dtype=jnp.int32)
        w0 = plsc.load_gather(w0_v, [jv])
        w1s = plsc.load_gather(w1_v, [jv])
        for c in range(D // 16):
            sl = pl.ds(c * 16, 16)
            r0_v[j, sl] = r0_v[j, sl] * w0 + r1_v[j, sl] * w1s
        return carry

    lax.fori_loop(0, 64, body, 0)
    pltpu.sync_copy(r0_v, o_h.at[pl.ds(tb, 64)])


def _combine(y, dst, rw_flat):
    f = pl.kernel(
        _combine_body,
        out_type=jax.ShapeDtypeStruct((T, D), jnp.float32),
        mesh=plsc.VectorSubcoreMesh(core_axis_name="c", subcore_axis_name="s"),
        compiler_params=pltpu.CompilerParams(needs_layout_passes=False),
        scratch_types=[
            pltpu.VMEM((64,), jnp.int32),
            pltpu.VMEM((64,), jnp.int32),
            pltpu.VMEM((64,), jnp.float32),
            pltpu.VMEM((64,), jnp.float32),
            pltpu.VMEM((64, D), jnp.float32),
            pltpu.VMEM((64, D), jnp.float32),
            pltpu.SemaphoreType.DMA,
            pltpu.SemaphoreType.DMA,
        ],
    )
    return f(y, dst, rw_flat)


def kernel(x, router_w, w1, w2):
    b, s, d = x.shape
    x2d = x.reshape(T, D)
    rwt = router_w.T
    sel_all, rank_all, rw_all, cnt0, offs, becol, actcol = _router(x2d, rwt)
    sel_flat = sel_all.reshape(NPAIR)
    rank_flat = rank_all.reshape(NPAIR)
    rw_flat = rw_all.reshape(NPAIR)
    xs, dst = _dispatch(sel_flat, rank_flat, cnt0.reshape(128),
                        offs.reshape(128), x2d)
    y = _ffn(becol[:NB, 0], actcol[:NB, 0], xs, w1, w2)
    out = _combine(y, dst, rw_flat)
    return out.reshape(b, s, d)


# speedup vs baseline: 1.1158x; 1.1158x over previous
import jax
import jax.numpy as jnp
from jax import lax
from jax.experimental import pallas as pl
from jax.experimental.pallas import tpu as pltpu
from jax.experimental.pallas import tpu_sc as plsc

E = 8
K = 2
D = 768
BS = 128
DFFN = 1536
T = 2048
NPAIR = T * K
NB = 40
P = NB * BS
TBLK = 128
NTB = T // TBLK
NW = 32


def _router_body(x_ref, rwin_ref, sel0_ref, sel1_ref, rank0_ref, rank1_ref,
                 rw0_ref, rw1_ref, c0_ref, offs_ref, be_ref, act_ref,
                 carry0, carry1):
    i = pl.program_id(0)

    @pl.when(i == 0)
    def _():
        carry0[...] = jnp.zeros_like(carry0)
        carry1[...] = jnp.zeros_like(carry1)

    xb = x_ref[...]
    logits = lax.dot_general(rwin_ref[...], xb, (((1,), (1,)), ((), ())),
                             preferred_element_type=jnp.float32)
    m = jnp.max(logits, axis=0, keepdims=True)
    ex = jnp.exp(logits - m)
    p = ex / jnp.sum(ex, axis=0, keepdims=True)
    sub8 = lax.broadcasted_iota(jnp.int32, (E, TBLK), 0)
    m1 = jnp.max(p, axis=0, keepdims=True)
    i1 = jnp.min(jnp.where(p >= m1, sub8, E), axis=0, keepdims=True)
    p2 = jnp.where(sub8 == i1, -1.0, p)
    m2 = jnp.max(p2, axis=0, keepdims=True)
    i2 = jnp.min(jnp.where(p2 >= m2, sub8, E), axis=0, keepdims=True)
    ssum = m1 + m2
    sel0_ref[...] = i1.reshape(TBLK)
    sel1_ref[...] = i2.reshape(TBLK)
    rw0_ref[...] = (m1 / ssum).reshape(TBLK)
    rw1_ref[...] = (m2 / ssum).reshape(TBLK)

    row128 = lax.broadcasted_iota(jnp.int32, (128, 128), 0)
    col128 = lax.broadcasted_iota(jnp.int32, (128, 128), 1)
    triu_incl = (row128 <= col128).astype(jnp.float32)
    for sel, carry, rref in ((i1, carry0, rank0_ref), (i2, carry1, rank1_ref)):
        oh = (row128 == sel).astype(jnp.float32)
        cum = jnp.dot(oh, triu_incl, preferred_element_type=jnp.float32)
        cb = carry[...]
        rank = jnp.sum(oh * (cum + cb - 1.0), axis=0, keepdims=True)
        rref[...] = rank.astype(jnp.int32).reshape(TBLK)
        carry[...] = cb + jnp.sum(oh, axis=1, keepdims=True)

    @pl.when(i == NTB - 1)
    def _():
        c0c = carry0[...]
        tot = (c0c + carry1[...]).astype(jnp.int32)
        padded = ((tot + 127) >> 7) << 7
        nblk = (padded >> 7).astype(jnp.float32)
        low_strict = (row128 > col128).astype(jnp.float32)
        low_incl = (row128 >= col128).astype(jnp.float32)
        offs_col = jnp.dot(low_strict, padded.astype(jnp.float32),
                           preferred_element_type=jnp.float32)
        bo_col = jnp.dot(low_incl, nblk, preferred_element_type=jnp.float32)
        eye = (row128 == col128).astype(jnp.float32)
        c0_ref[...] = jnp.sum(eye * c0c, axis=0).astype(jnp.int32)
        offs_ref[...] = jnp.sum(eye * offs_col, axis=0).astype(jnp.int32)
        lane_row = col128[0:1, :].astype(jnp.float32)
        cmp = jnp.logical_and(bo_col <= lane_row, row128 < E)
        be_row = jnp.sum(cmp.astype(jnp.float32), axis=0, keepdims=True)
        be_ref[...] = jnp.minimum(be_row, float(E - 1)).astype(
            jnp.int32).reshape(128)
        nbtot = jnp.sum(jnp.where(row128[:, 0:1] < E, nblk, 0.0), axis=0,
                        keepdims=True)
        act_ref[...] = (lane_row < nbtot).astype(jnp.int32).reshape(128)


def _router(x2d, router_w):
    flat_i = jax.ShapeDtypeStruct((T,), jnp.int32)
    flat_f = jax.ShapeDtypeStruct((T,), jnp.float32)
    meta_i = jax.ShapeDtypeStruct((128,), jnp.int32)
    blk = pl.BlockSpec((TBLK,), lambda i: (i,))
    meta = pl.BlockSpec((128,), lambda i: (0,))
    return pl.pallas_call(
        _router_body,
        grid=(NTB,),
        in_specs=[
            pl.BlockSpec((TBLK, D), lambda i: (i, 0)),
            pl.BlockSpec((E, D), lambda i: (0, 0)),
        ],
        out_specs=[blk, blk, blk, blk, blk, blk, meta, meta, meta, meta],
        out_shape=[flat_i, flat_i, flat_i, flat_i, flat_f, flat_f,
                   meta_i, meta_i, meta_i, meta_i],
        scratch_shapes=[
            pltpu.VMEM((128, 1), jnp.float32),
            pltpu.VMEM((128, 1), jnp.float32),
        ],
    )(x2d, router_w)


def _dispatch_body(sel0_h, sel1_h, rank0_h, rank1_h, c0_h, offs_h, x_h,
                   xs_h, dst_h, selc_v, rankc_v, c0_v, offs_v, dst_v, tok_v,
                   rows_v, sem1, sem2):
    wid = lax.axis_index("s") * 2 + lax.axis_index("c")
    kflag = wid // 16
    tb = (wid % 16) * 128
    pb = wid * 128

    @pl.when(kflag == 0)
    def _():
        pltpu.sync_copy(sel0_h.at[pl.ds(tb, 128)], selc_v)
        pltpu.sync_copy(rank0_h.at[pl.ds(tb, 128)], rankc_v)

    @pl.when(kflag == 1)
    def _():
        pltpu.sync_copy(sel1_h.at[pl.ds(tb, 128)], selc_v)
        pltpu.sync_copy(rank1_h.at[pl.ds(tb, 128)], rankc_v)

    pltpu.sync_copy(c0_h.at[pl.ds(0, 16)], c0_v)
    pltpu.sync_copy(offs_h.at[pl.ds(0, 16)], offs_v)
    kvec = jnp.full((16,), kflag, dtype=jnp.int32)
    for j in range(8):
        s16 = selc_v[pl.ds(j * 16, 16)]
        r16 = rankc_v[pl.ds(j * 16, 16)]
        o16 = plsc.load_gather(offs_v, [s16])
        c016 = plsc.load_gather(c0_v, [s16])
        d16 = o16 + c016 * kvec + r16
        dst_v[pl.ds(j * 16, 16)] = d16
        tok_v[pl.ds(j * 16, 16)] = tb + j * 16 + lax.iota(jnp.int32, 16)
    pltpu.async_copy(x_h.at[tok_v], rows_v, sem1).wait()
    pltpu.async_copy(rows_v, xs_h.at[dst_v], sem2).wait()
    pltpu.sync_copy(dst_v, dst_h.at[pl.ds(pb, 128)])


def _dispatch(sel0, sel1, rank0, rank1, c0v, offsv, x2d):
    f = pl.kernel(
        _dispatch_body,
        out_type=[
            jax.ShapeDtypeStruct((P, D), jnp.float32),
            jax.ShapeDtypeStruct((NPAIR,), jnp.int32),
        ],
        mesh=plsc.VectorSubcoreMesh(core_axis_name="c", subcore_axis_name="s"),
        compiler_params=pltpu.CompilerParams(needs_layout_passes=False),
        scratch_types=[
            pltpu.VMEM((128,), jnp.int32),
            pltpu.VMEM((128,), jnp.int32),
            pltpu.VMEM((16,), jnp.int32),
            pltpu.VMEM((16,), jnp.int32),
            pltpu.VMEM((128,), jnp.int32),
            pltpu.VMEM((128,), jnp.int32),
            pltpu.VMEM((128, D), jnp.float32),
            pltpu.SemaphoreType.DMA,
            pltpu.SemaphoreType.DMA,
        ],
    )
    return f(sel0, sel1, rank0, rank1, c0v, offsv, x2d)


def _ffn_body(be_ref, act_ref, xs_ref, w1_ref, w2_ref, y_ref):
    b = pl.program_id(0)

    @pl.when(act_ref[b] == 1)
    def _():
        h = jnp.dot(xs_ref[...].astype(jnp.bfloat16),
                    w1_ref[...].astype(jnp.bfloat16),
                    preferred_element_type=jnp.float32)
        h = jax.nn.gelu(h)
        y_ref[...] = jnp.dot(h.astype(jnp.bfloat16),
                             w2_ref[...].astype(jnp.bfloat16),
                             preferred_element_type=jnp.float32)


def _ffn(be, act, xs, w1, w2):
    grid_spec = pltpu.PrefetchScalarGridSpec(
        num_scalar_prefetch=2,
        grid=(NB,),
        in_specs=[
            pl.BlockSpec((BS, D), lambda b, be_r, act_r: (b, 0)),
            pl.BlockSpec((D, DFFN), lambda b, be_r, act_r: (0, be_r[b])),
            pl.BlockSpec((DFFN, D), lambda b, be_r, act_r: (be_r[b], 0)),
        ],
        out_specs=pl.BlockSpec((BS, D), lambda b, be_r, act_r: (b, 0)),
    )
    return pl.pallas_call(
        _ffn_body,
        grid_spec=grid_spec,
        out_shape=jax.ShapeDtypeStruct((P, D), jnp.float32),
    )(be, act, xs, w1, w2)


def _combine_body(y_h, dst_h, rw0_h, rw1_h, o_h, i0_v, i1_v, w0_v, w1_v,
                  r0_v, r1_v, sem1, sem2):
    wid = lax.axis_index("s") * 2 + lax.axis_index("c")
    tb = wid * 64
    pltpu.sync_copy(dst_h.at[pl.ds(tb, 64)], i0_v)
    pltpu.sync_copy(dst_h.at[pl.ds(T + tb, 64)], i1_v)
    pltpu.sync_copy(rw0_h.at[pl.ds(tb, 64)], w0_v)
    pltpu.sync_copy(rw1_h.at[pl.ds(tb, 64)], w1_v)
    pltpu.async_copy(y_h.at[i0_v], r0_v, sem1).wait()
    pltpu.async_copy(y_h.at[i1_v], r1_v, sem2).wait()

    def body(j, carry):
        jv = jnp.full((16,), j, dtype=jnp.int32)
        w0 = plsc.load_gather(w0_v, [jv])
        w1s = plsc.load_gather(w1_v, [jv])
        for c in range(D // 16):
            sl = pl.ds(c * 16, 16)
            r0_v[j, sl] = r0_v[j, sl] * w0 + r1_v[j, sl] * w1s
        return carry

    lax.fori_loop(0, 64, body, 0)
    pltpu.sync_copy(r0_v, o_h.at[pl.ds(tb, 64)])


def _combine(y, dst, rw0f, rw1f):
    f = pl.kernel(
        _combine_body,
        out_type=jax.ShapeDtypeStruct((T, D), jnp.float32),
        mesh=plsc.VectorSubcoreMesh(core_axis_name="c", subcore_axis_name="s"),
        compiler_params=pltpu.CompilerParams(needs_layout_passes=False),
        scratch_types=[
            pltpu.VMEM((64,), jnp.int32),
            pltpu.VMEM((64,), jnp.int32),
            pltpu.VMEM((64,), jnp.float32),
            pltpu.VMEM((64,), jnp.float32),
            pltpu.VMEM((64, D), jnp.float32),
            pltpu.VMEM((64, D), jnp.float32),
            pltpu.SemaphoreType.DMA,
            pltpu.SemaphoreType.DMA,
        ],
    )
    return f(y, dst, rw0f, rw1f)


def kernel(x, router_w, w1, w2):
    b, s, d = x.shape
    x2d = x.reshape(T, D)
    (sel0, sel1, rank0, rank1, rw0f, rw1f,
     c0v, offsv, bev, actv) = _router(x2d, router_w)
    xs, dst = _dispatch(sel0, sel1, rank0, rank1, c0v, offsv, x2d)
    y = _ffn(bev, actv, xs, w1, w2)
    out = _combine(y, dst, rw0f, rw1f)
    return out.reshape(b, s, d)


# speedup vs baseline: 1.1478x; 1.0287x over previous
import jax
import jax.numpy as jnp
from jax import lax
from jax.experimental import pallas as pl
from jax.experimental.pallas import tpu as pltpu
from jax.experimental.pallas import tpu_sc as plsc

E = 8
K = 2
D = 768
BS = 128
DFFN = 1536
T = 2048
NPAIR = T * K
NB = 40
P = NB * BS
TBLK = 128
NTB = T // TBLK
NW = 32


def _router_body(x_ref, rwin_ref, sel0_ref, sel1_ref, rank0_ref, rank1_ref,
                 rw0_ref, rw1_ref, c0_ref, offs_ref, nblk_ref,
                 carry0, carry1):
    i = pl.program_id(0)

    @pl.when(i == 0)
    def _():
        carry0[...] = jnp.zeros_like(carry0)
        carry1[...] = jnp.zeros_like(carry1)

    xb = x_ref[...]
    logits = lax.dot_general(rwin_ref[...], xb, (((1,), (1,)), ((), ())),
                             preferred_element_type=jnp.float32)
    m = jnp.max(logits, axis=0, keepdims=True)
    ex = jnp.exp(logits - m)
    p = ex / jnp.sum(ex, axis=0, keepdims=True)
    sub8 = lax.broadcasted_iota(jnp.int32, (E, TBLK), 0)
    m1 = jnp.max(p, axis=0, keepdims=True)
    i1 = jnp.min(jnp.where(p >= m1, sub8, E), axis=0, keepdims=True)
    p2 = jnp.where(sub8 == i1, -1.0, p)
    m2 = jnp.max(p2, axis=0, keepdims=True)
    i2 = jnp.min(jnp.where(p2 >= m2, sub8, E), axis=0, keepdims=True)
    ssum = m1 + m2
    sel0_ref[...] = i1.reshape(TBLK)
    sel1_ref[...] = i2.reshape(TBLK)
    rw0_ref[...] = (m1 / ssum).reshape(TBLK)
    rw1_ref[...] = (m2 / ssum).reshape(TBLK)

    row128 = lax.broadcasted_iota(jnp.int32, (128, 128), 0)
    col128 = lax.broadcasted_iota(jnp.int32, (128, 128), 1)
    triu_incl = (row128 <= col128).astype(jnp.float32)
    for sel, carry, rref in ((i1, carry0, rank0_ref), (i2, carry1, rank1_ref)):
        oh = (row128 == sel).astype(jnp.float32)
        cum = jnp.dot(oh, triu_incl, preferred_element_type=jnp.float32)
        cb = carry[...]
        rank = jnp.sum(oh * (cum + cb - 1.0), axis=0, keepdims=True)
        rref[...] = rank.astype(jnp.int32).reshape(TBLK)
        carry[...] = cb + jnp.sum(oh, axis=1, keepdims=True)

    @pl.when(i == NTB - 1)
    def _():
        c0c = carry0[...]
        tot = (c0c + carry1[...]).astype(jnp.int32)
        padded = ((tot + 127) >> 7) << 7
        nblk = (padded >> 7).astype(jnp.float32)
        low_strict = (row128 > col128).astype(jnp.float32)
        offs_col = jnp.dot(low_strict, padded.astype(jnp.float32),
                           preferred_element_type=jnp.float32)
        eye = (row128 == col128).astype(jnp.float32)
        c0_ref[...] = jnp.sum(eye * c0c, axis=0).astype(jnp.int32)
        offs_ref[...] = jnp.sum(eye * offs_col, axis=0).astype(jnp.int32)
        nblk_ref[...] = jnp.sum(eye * nblk, axis=0).astype(jnp.int32)


def _router(x2d, router_w):
    flat_i = jax.ShapeDtypeStruct((T,), jnp.int32)
    flat_f = jax.ShapeDtypeStruct((T,), jnp.float32)
    meta_i = jax.ShapeDtypeStruct((128,), jnp.int32)
    blk = pl.BlockSpec((TBLK,), lambda i: (i,))
    meta = pl.BlockSpec((128,), lambda i: (0,))
    return pl.pallas_call(
        _router_body,
        grid=(NTB,),
        in_specs=[
            pl.BlockSpec((TBLK, D), lambda i: (i, 0)),
            pl.BlockSpec((E, D), lambda i: (0, 0)),
        ],
        out_specs=[blk, blk, blk, blk, blk, blk, meta, meta, meta],
        out_shape=[flat_i, flat_i, flat_i, flat_i, flat_f, flat_f,
                   meta_i, meta_i, meta_i],
        scratch_shapes=[
            pltpu.VMEM((128, 1), jnp.float32),
            pltpu.VMEM((128, 1), jnp.float32),
        ],
    )(x2d, router_w)


def _dispatch_body(sel0_h, sel1_h, rank0_h, rank1_h, c0_h, offs_h, x_h,
                   xs_h, dst_h, selc_v, rankc_v, c0_v, offs_v, dst_v, tok_v,
                   rows_v, sem1, sem2):
    wid = lax.axis_index("s") * 2 + lax.axis_index("c")
    kflag = wid // 16
    tb = (wid % 16) * 128
    pb = wid * 128

    @pl.when(kflag == 0)
    def _():
        pltpu.sync_copy(sel0_h.at[pl.ds(tb, 128)], selc_v)
        pltpu.sync_copy(rank0_h.at[pl.ds(tb, 128)], rankc_v)

    @pl.when(kflag == 1)
    def _():
        pltpu.sync_copy(sel1_h.at[pl.ds(tb, 128)], selc_v)
        pltpu.sync_copy(rank1_h.at[pl.ds(tb, 128)], rankc_v)

    pltpu.sync_copy(c0_h.at[pl.ds(0, 16)], c0_v)
    pltpu.sync_copy(offs_h.at[pl.ds(0, 16)], offs_v)
    kvec = jnp.full((16,), kflag, dtype=jnp.int32)
    for j in range(8):
        s16 = selc_v[pl.ds(j * 16, 16)]
        r16 = rankc_v[pl.ds(j * 16, 16)]
        o16 = plsc.load_gather(offs_v, [s16])
        c016 = plsc.load_gather(c0_v, [s16])
        d16 = o16 + c016 * kvec + r16
        dst_v[pl.ds(j * 16, 16)] = d16
        tok_v[pl.ds(j * 16, 16)] = tb + j * 16 + lax.iota(jnp.int32, 16)
    pltpu.async_copy(x_h.at[tok_v], rows_v, sem1).wait()
    pltpu.async_copy(rows_v, xs_h.at[dst_v], sem2).wait()
    pltpu.sync_copy(dst_v, dst_h.at[pl.ds(pb, 128)])


def _dispatch(sel0, sel1, rank0, rank1, c0v, offsv, x2d):
    f = pl.kernel(
        _dispatch_body,
        out_type=[
            jax.ShapeDtypeStruct((P, D), jnp.float32),
            jax.ShapeDtypeStruct((NPAIR,), jnp.int32),
        ],
        mesh=plsc.VectorSubcoreMesh(core_axis_name="c", subcore_axis_name="s"),
        compiler_params=pltpu.CompilerParams(needs_layout_passes=False),
        scratch_types=[
            pltpu.VMEM((128,), jnp.int32),
            pltpu.VMEM((128,), jnp.int32),
            pltpu.VMEM((16,), jnp.int32),
            pltpu.VMEM((16,), jnp.int32),
            pltpu.VMEM((128,), jnp.int32),
            pltpu.VMEM((128,), jnp.int32),
            pltpu.VMEM((128, D), jnp.float32),
            pltpu.SemaphoreType.DMA,
            pltpu.SemaphoreType.DMA,
        ],
    )
    return f(sel0, sel1, rank0, rank1, c0v, offsv, x2d)


def _ffn_body(nb_ref, xs_hbm, w1_hbm, w2_hbm, y_hbm, w1b, w2b, xsb, yb,
              w1s, w2s, xss, yss):
    def w1cp(e, slot):
        return pltpu.make_async_copy(
            w1_hbm.at[:, pl.ds(e * DFFN, DFFN)], w1b.at[slot], w1s.at[slot])

    def w2cp(e, slot):
        return pltpu.make_async_copy(
            w2_hbm.at[pl.ds(e * DFFN, DFFN), :], w2b.at[slot], w2s.at[slot])

    def xscp(g, slot):
        return pltpu.make_async_copy(
            xs_hbm.at[pl.ds(g * BS, BS)], xsb.at[slot], xss.at[slot])

    def ycp(g, slot):
        return pltpu.make_async_copy(
            yb.at[slot], y_hbm.at[pl.ds(g * BS, BS)], yss.at[slot])

    nbtot = nb_ref[0]
    for e in range(1, E):
        nbtot = nbtot + nb_ref[e]

    for e in range(3):
        w1cp(e, e).start()
        w2cp(e, e).start()
    xscp(0, 0).start()

    g = 0
    for e in range(E):
        slot = e % 3
        w1cp(e, slot).wait()
        w2cp(e, slot).wait()

        def blk(j, g, slot=slot):
            xslot = lax.rem(g, 2)
            xscp(g, xslot).wait()

            @pl.when(g + 1 < nbtot)
            def _():
                xscp(g + 1, lax.rem(g + 1, 2)).start()

            @pl.when(g >= 2)
            def _():
                ycp(g - 2, xslot).wait()

            h = jnp.dot(xsb[xslot].astype(jnp.bfloat16),
                        w1b[slot].astype(jnp.bfloat16),
                        preferred_element_type=jnp.float32)
            h = jax.nn.gelu(h)
            yb[xslot] = jnp.dot(h.astype(jnp.bfloat16),
                                w2b[slot].astype(jnp.bfloat16),
                                preferred_element_type=jnp.float32)
            ycp(g, xslot).start()
            return g + 1

        g = lax.fori_loop(0, nb_ref[e], blk, g)
        if e + 3 < E:
            w1cp(e + 3, slot).start()
            w2cp(e + 3, slot).start()

    ycp(g - 1, lax.rem(g - 1, 2)).wait()
    ycp(g - 2, lax.rem(g - 2, 2)).wait()


def _ffn(nblk, xs, w1, w2):
    return pl.pallas_call(
        _ffn_body,
        in_specs=[
            pl.BlockSpec(memory_space=pltpu.SMEM),
            pl.BlockSpec(memory_space=pl.ANY),
            pl.BlockSpec(memory_space=pl.ANY),
            pl.BlockSpec(memory_space=pl.ANY),
        ],
        out_specs=pl.BlockSpec(memory_space=pl.ANY),
        out_shape=jax.ShapeDtypeStruct((P, D), jnp.float32),
        scratch_shapes=[
            pltpu.VMEM((3, D, DFFN), jnp.float32),
            pltpu.VMEM((3, DFFN, D), jnp.float32),
            pltpu.VMEM((2, BS, D), jnp.float32),
            pltpu.VMEM((2, BS, D), jnp.float32),
            pltpu.SemaphoreType.DMA((3,)),
            pltpu.SemaphoreType.DMA((3,)),
            pltpu.SemaphoreType.DMA((2,)),
            pltpu.SemaphoreType.DMA((2,)),
        ],
    )(nblk, xs, w1, w2)


def _combine_body(y_h, dst_h, rw0_h, rw1_h, o_h, i0_v, i1_v, w0_v, w1_v,
                  r0_v, r1_v, sem1, sem2):
    wid = lax.axis_index("s") * 2 + lax.axis_index("c")
    tb = wid * 64
    pltpu.sync_copy(dst_h.at[pl.ds(tb, 64)], i0_v)
    pltpu.sync_copy(dst_h.at[pl.ds(T + tb, 64)], i1_v)
    pltpu.sync_copy(rw0_h.at[pl.ds(tb, 64)], w0_v)
    pltpu.sync_copy(rw1_h.at[pl.ds(tb, 64)], w1_v)
    pltpu.async_copy(y_h.at[i0_v], r0_v, sem1).wait()
    pltpu.async_copy(y_h.at[i1_v], r1_v, sem2).wait()

    def body(j, carry):
        jv = jnp.full((16,), j, dtype=jnp.int32)
        w0 = plsc.load_gather(w0_v, [jv])
        w1s = plsc.load_gather(w1_v, [jv])
        for c in range(D // 16):
            sl = pl.ds(c * 16, 16)
            r0_v[j, sl] = r0_v[j, sl] * w0 + r1_v[j, sl] * w1s
        return carry

    lax.fori_loop(0, 64, body, 0)
    pltpu.sync_copy(r0_v, o_h.at[pl.ds(tb, 64)])


def _combine(y, dst, rw0f, rw1f):
    f = pl.kernel(
        _combine_body,
        out_type=jax.ShapeDtypeStruct((T, D), jnp.float32),
        mesh=plsc.VectorSubcoreMesh(core_axis_name="c", subcore_axis_name="s"),
        compiler_params=pltpu.CompilerParams(needs_layout_passes=False),
        scratch_types=[
            pltpu.VMEM((64,), jnp.int32),
            pltpu.VMEM((64,), jnp.int32),
            pltpu.VMEM((64,), jnp.float32),
            pltpu.VMEM((64,), jnp.float32),
            pltpu.VMEM((64, D), jnp.float32),
            pltpu.VMEM((64, D), jnp.float32),
            pltpu.SemaphoreType.DMA,
            pltpu.SemaphoreType.DMA,
        ],
    )
    return f(y, dst, rw0f, rw1f)


def kernel(x, router_w, w1, w2):
    b, s, d = x.shape
    x2d = x.reshape(T, D)
    (sel0, sel1, rank0, rank1, rw0f, rw1f,
     c0v, offsv, nblkv) = _router(x2d, router_w)
    xs, dst = _dispatch(sel0, sel1, rank0, rank1, c0v, offsv, x2d)
    y = _ffn(nblkv, xs, w1, w2)
    out = _combine(y, dst, rw0f, rw1f)
    return out.reshape(b, s, d)


# speedup vs baseline: 1.2154x; 1.0589x over previous
import jax
import jax.numpy as jnp
from jax import lax
from jax.experimental import pallas as pl
from jax.experimental.pallas import tpu as pltpu
from jax.experimental.pallas import tpu_sc as plsc

E = 8
K = 2
D = 768
BS = 256
DFFN = 1536
T = 2048
NPAIR = T * K
NB = 23
P = NB * BS
TBLK = 128
NTB = T // TBLK
NW = 32


def _router_body(x_ref, rwin_ref, sel0_ref, sel1_ref, rank0_ref, rank1_ref,
                 rw0_ref, rw1_ref, c0_ref, offs_ref, nblk_ref,
                 carry0, carry1):
    i = pl.program_id(0)

    @pl.when(i == 0)
    def _():
        carry0[...] = jnp.zeros_like(carry0)
        carry1[...] = jnp.zeros_like(carry1)

    xb = x_ref[...]
    logits = lax.dot_general(rwin_ref[...], xb, (((1,), (1,)), ((), ())),
                             preferred_element_type=jnp.float32)
    m = jnp.max(logits, axis=0, keepdims=True)
    ex = jnp.exp(logits - m)
    p = ex / jnp.sum(ex, axis=0, keepdims=True)
    sub8 = lax.broadcasted_iota(jnp.int32, (E, TBLK), 0)
    m1 = jnp.max(p, axis=0, keepdims=True)
    i1 = jnp.min(jnp.where(p >= m1, sub8, E), axis=0, keepdims=True)
    p2 = jnp.where(sub8 == i1, -1.0, p)
    m2 = jnp.max(p2, axis=0, keepdims=True)
    i2 = jnp.min(jnp.where(p2 >= m2, sub8, E), axis=0, keepdims=True)
    ssum = m1 + m2
    sel0_ref[...] = i1.reshape(TBLK)
    sel1_ref[...] = i2.reshape(TBLK)
    rw0_ref[...] = (m1 / ssum).reshape(TBLK)
    rw1_ref[...] = (m2 / ssum).reshape(TBLK)

    row128 = lax.broadcasted_iota(jnp.int32, (128, 128), 0)
    col128 = lax.broadcasted_iota(jnp.int32, (128, 128), 1)
    triu_incl = (row128 <= col128).astype(jnp.float32)
    for sel, carry, rref in ((i1, carry0, rank0_ref), (i2, carry1, rank1_ref)):
        oh = (row128 == sel).astype(jnp.float32)
        cum = jnp.dot(oh, triu_incl, preferred_element_type=jnp.float32)
        cb = carry[...]
        rank = jnp.sum(oh * (cum + cb - 1.0), axis=0, keepdims=True)
        rref[...] = rank.astype(jnp.int32).reshape(TBLK)
        carry[...] = cb + jnp.sum(oh, axis=1, keepdims=True)

    @pl.when(i == NTB - 1)
    def _():
        c0c = carry0[...]
        tot = (c0c + carry1[...]).astype(jnp.int32)
        padded = ((tot + BS - 1) >> 8) << 8
        nblk = (padded >> 8).astype(jnp.float32)
        low_strict = (row128 > col128).astype(jnp.float32)
        offs_col = jnp.dot(low_strict, padded.astype(jnp.float32),
                           preferred_element_type=jnp.float32)
        eye = (row128 == col128).astype(jnp.float32)
        c0_ref[...] = jnp.sum(eye * c0c, axis=0).astype(jnp.int32)
        offs_ref[...] = jnp.sum(eye * offs_col, axis=0).astype(jnp.int32)
        nblk_ref[...] = jnp.sum(eye * nblk, axis=0).astype(jnp.int32)


def _router(x2d, router_w):
    flat_i = jax.ShapeDtypeStruct((T,), jnp.int32)
    flat_f = jax.ShapeDtypeStruct((T,), jnp.float32)
    meta_i = jax.ShapeDtypeStruct((128,), jnp.int32)
    blk = pl.BlockSpec((TBLK,), lambda i: (i,))
    meta = pl.BlockSpec((128,), lambda i: (0,))
    return pl.pallas_call(
        _router_body,
        grid=(NTB,),
        in_specs=[
            pl.BlockSpec((TBLK, D), lambda i: (i, 0)),
            pl.BlockSpec((E, D), lambda i: (0, 0)),
        ],
        out_specs=[blk, blk, blk, blk, blk, blk, meta, meta, meta],
        out_shape=[flat_i, flat_i, flat_i, flat_i, flat_f, flat_f,
                   meta_i, meta_i, meta_i],
        scratch_shapes=[
            pltpu.VMEM((128, 1), jnp.float32),
            pltpu.VMEM((128, 1), jnp.float32),
        ],
    )(x2d, router_w)


def _dispatch_body(sel0_h, sel1_h, rank0_h, rank1_h, c0_h, offs_h, x_h,
                   xs_h, dst_h, selc_v, rankc_v, c0_v, offs_v, dst_v, tok_v,
                   rows_v, sem1, sem2):
    wid = lax.axis_index("s") * 2 + lax.axis_index("c")
    kflag = wid // 16
    tb = (wid % 16) * 128
    pb = wid * 128

    @pl.when(kflag == 0)
    def _():
        pltpu.sync_copy(sel0_h.at[pl.ds(tb, 128)], selc_v)
        pltpu.sync_copy(rank0_h.at[pl.ds(tb, 128)], rankc_v)

    @pl.when(kflag == 1)
    def _():
        pltpu.sync_copy(sel1_h.at[pl.ds(tb, 128)], selc_v)
        pltpu.sync_copy(rank1_h.at[pl.ds(tb, 128)], rankc_v)

    pltpu.sync_copy(c0_h.at[pl.ds(0, 16)], c0_v)
    pltpu.sync_copy(offs_h.at[pl.ds(0, 16)], offs_v)
    kvec = jnp.full((16,), kflag, dtype=jnp.int32)
    for j in range(8):
        s16 = selc_v[pl.ds(j * 16, 16)]
        r16 = rankc_v[pl.ds(j * 16, 16)]
        o16 = plsc.load_gather(offs_v, [s16])
        c016 = plsc.load_gather(c0_v, [s16])
        d16 = o16 + c016 * kvec + r16
        dst_v[pl.ds(j * 16, 16)] = d16
        tok_v[pl.ds(j * 16, 16)] = tb + j * 16 + lax.iota(jnp.int32, 16)
    pltpu.async_copy(x_h.at[tok_v], rows_v, sem1).wait()
    pltpu.async_copy(rows_v, xs_h.at[dst_v], sem2).wait()
    pltpu.sync_copy(dst_v, dst_h.at[pl.ds(pb, 128)])


def _dispatch(sel0, sel1, rank0, rank1, c0v, offsv, x2d):
    f = pl.kernel(
        _dispatch_body,
        out_type=[
            jax.ShapeDtypeStruct((P, D), jnp.float32),
            jax.ShapeDtypeStruct((NPAIR,), jnp.int32),
        ],
        mesh=plsc.VectorSubcoreMesh(core_axis_name="c", subcore_axis_name="s"),
        compiler_params=pltpu.CompilerParams(needs_layout_passes=False),
        scratch_types=[
            pltpu.VMEM((128,), jnp.int32),
            pltpu.VMEM((128,), jnp.int32),
            pltpu.VMEM((16,), jnp.int32),
            pltpu.VMEM((16,), jnp.int32),
            pltpu.VMEM((128,), jnp.int32),
            pltpu.VMEM((128,), jnp.int32),
            pltpu.VMEM((128, D), jnp.float32),
            pltpu.SemaphoreType.DMA,
            pltpu.SemaphoreType.DMA,
        ],
    )
    return f(sel0, sel1, rank0, rank1, c0v, offsv, x2d)


def _ffn_body(nb_ref, xs_hbm, w1_hbm, w2_hbm, y_hbm, w1b, w2b, xsb, yb,
              w1s, w2s, xss, yss):
    def w1cp(e, slot):
        return pltpu.make_async_copy(
            w1_hbm.at[:, pl.ds(e * DFFN, DFFN)], w1b.at[slot], w1s.at[slot])

    def w2cp(e, slot):
        return pltpu.make_async_copy(
            w2_hbm.at[pl.ds(e * DFFN, DFFN), :], w2b.at[slot], w2s.at[slot])

    def xscp(g, slot):
        return pltpu.make_async_copy(
            xs_hbm.at[pl.ds(g * BS, BS)], xsb.at[slot], xss.at[slot])

    def ycp(g, slot):
        return pltpu.make_async_copy(
            yb.at[slot], y_hbm.at[pl.ds(g * BS, BS)], yss.at[slot])

    nbtot = nb_ref[0]
    for e in range(1, E):
        nbtot = nbtot + nb_ref[e]

    for e in range(3):
        w1cp(e, e).start()
        w2cp(e, e).start()
    xscp(0, 0).start()

    g = 0
    for e in range(E):
        slot = e % 3
        w1cp(e, slot).wait()
        w2cp(e, slot).wait()

        def blk(j, g, slot=slot):
            xslot = lax.rem(g, 2)
            xscp(g, xslot).wait()

            @pl.when(g + 1 < nbtot)
            def _():
                xscp(g + 1, lax.rem(g + 1, 2)).start()

            @pl.when(g >= 2)
            def _():
                ycp(g - 2, xslot).wait()

            h = jnp.dot(xsb[xslot].astype(jnp.bfloat16),
                        w1b[slot].astype(jnp.bfloat16),
                        preferred_element_type=jnp.float32)
            h = jax.nn.gelu(h)
            yb[xslot] = jnp.dot(h.astype(jnp.bfloat16),
                                w2b[slot].astype(jnp.bfloat16),
                                preferred_element_type=jnp.float32)
            ycp(g, xslot).start()
            return g + 1

        g = lax.fori_loop(0, nb_ref[e], blk, g)
        if e + 3 < E:
            w1cp(e + 3, slot).start()
            w2cp(e + 3, slot).start()

    ycp(g - 1, lax.rem(g - 1, 2)).wait()
    ycp(g - 2, lax.rem(g - 2, 2)).wait()


def _ffn(nblk, xs, w1, w2):
    return pl.pallas_call(
        _ffn_body,
        in_specs=[
            pl.BlockSpec(memory_space=pltpu.SMEM),
            pl.BlockSpec(memory_space=pl.ANY),
            pl.BlockSpec(memory_space=pl.ANY),
            pl.BlockSpec(memory_space=pl.ANY),
        ],
        out_specs=pl.BlockSpec(memory_space=pl.ANY),
        out_shape=jax.ShapeDtypeStruct((P, D), jnp.float32),
        scratch_shapes=[
            pltpu.VMEM((3, D, DFFN), jnp.float32),
            pltpu.VMEM((3, DFFN, D), jnp.float32),
            pltpu.VMEM((2, BS, D), jnp.float32),
            pltpu.VMEM((2, BS, D), jnp.float32),
            pltpu.SemaphoreType.DMA((3,)),
            pltpu.SemaphoreType.DMA((3,)),
            pltpu.SemaphoreType.DMA((2,)),
            pltpu.SemaphoreType.DMA((2,)),
        ],
    )(nblk, xs, w1, w2)


def _combine_body(y_h, dst_h, rw0_h, rw1_h, o_h, i0_v, i1_v, w0_v, w1_v,
                  r0_v, r1_v, sem1, sem2):
    wid = lax.axis_index("s") * 2 + lax.axis_index("c")
    tb = wid * 64
    pltpu.sync_copy(dst_h.at[pl.ds(tb, 64)], i0_v)
    pltpu.sync_copy(dst_h.at[pl.ds(T + tb, 64)], i1_v)
    pltpu.sync_copy(rw0_h.at[pl.ds(tb, 64)], w0_v)
    pltpu.sync_copy(rw1_h.at[pl.ds(tb, 64)], w1_v)
    pltpu.async_copy(y_h.at[i0_v], r0_v, sem1).wait()
    pltpu.async_copy(y_h.at[i1_v], r1_v, sem2).wait()

    def body(j, carry):
        jv = jnp.full((16,), j, dtype=jnp.int32)
        w0 = plsc.load_gather(w0_v, [jv])
        w1s = plsc.load_gather(w1_v, [jv])
        for c in range(D // 16):
            sl = pl.ds(c * 16, 16)
            r0_v[j, sl] = r0_v[j, sl] * w0 + r1_v[j, sl] * w1s
        return carry

    lax.fori_loop(0, 64, body, 0)
    pltpu.sync_copy(r0_v, o_h.at[pl.ds(tb, 64)])


def _combine(y, dst, rw0f, rw1f):
    f = pl.kernel(
        _combine_body,
        out_type=jax.ShapeDtypeStruct((T, D), jnp.float32),
        mesh=plsc.VectorSubcoreMesh(core_axis_name="c", subcore_axis_name="s"),
        compiler_params=pltpu.CompilerParams(needs_layout_passes=False),
        scratch_types=[
            pltpu.VMEM((64,), jnp.int32),
            pltpu.VMEM((64,), jnp.int32),
            pltpu.VMEM((64,), jnp.float32),
            pltpu.VMEM((64,), jnp.float32),
            pltpu.VMEM((64, D), jnp.float32),
            pltpu.VMEM((64, D), jnp.float32),
            pltpu.SemaphoreType.DMA,
            pltpu.SemaphoreType.DMA,
        ],
    )
    return f(y, dst, rw0f, rw1f)


def kernel(x, router_w, w1, w2):
    b, s, d = x.shape
    x2d = x.reshape(T, D)
    (sel0, sel1, rank0, rank1, rw0f, rw1f,
     c0v, offsv, nblkv) = _router(x2d, router_w)
    xs, dst = _dispatch(sel0, sel1, rank0, rank1, c0v, offsv, x2d)
    y = _ffn(nblkv, xs, w1, w2)
    out = _combine(y, dst, rw0f, rw1f)
    return out.reshape(b, s, d)


# speedup vs baseline: 1.2206x; 1.0043x over previous
import jax
import jax.numpy as jnp
from jax import lax
from jax.experimental import pallas as pl
from jax.experimental.pallas import tpu as pltpu
from jax.experimental.pallas import tpu_sc as plsc

E = 8
K = 2
D = 768
BS = 256
DFFN = 1536
T = 2048
NPAIR = T * K
NB = 23
P = NB * BS
TBLK = 128
NTB = T // TBLK
NW = 32


def _router_body(x_ref, rwin_ref, sel0_ref, sel1_ref, rank0_ref, rank1_ref,
                 rw0_ref, rw1_ref, c0_ref, offs_ref, nblk_ref,
                 carry0, carry1):
    i = pl.program_id(0)

    @pl.when(i == 0)
    def _():
        carry0[...] = jnp.zeros_like(carry0)
        carry1[...] = jnp.zeros_like(carry1)

    xb = x_ref[...]
    logits = lax.dot_general(rwin_ref[...], xb, (((1,), (1,)), ((), ())),
                             preferred_element_type=jnp.float32)
    m = jnp.max(logits, axis=0, keepdims=True)
    ex = jnp.exp(logits - m)
    p = ex / jnp.sum(ex, axis=0, keepdims=True)
    sub8 = lax.broadcasted_iota(jnp.int32, (E, TBLK), 0)
    m1 = jnp.max(p, axis=0, keepdims=True)
    i1 = jnp.min(jnp.where(p >= m1, sub8, E), axis=0, keepdims=True)
    p2 = jnp.where(sub8 == i1, -1.0, p)
    m2 = jnp.max(p2, axis=0, keepdims=True)
    i2 = jnp.min(jnp.where(p2 >= m2, sub8, E), axis=0, keepdims=True)
    ssum = m1 + m2
    sel0_ref[...] = i1.reshape(TBLK)
    sel1_ref[...] = i2.reshape(TBLK)
    rw0_ref[...] = (m1 / ssum).reshape(TBLK)
    rw1_ref[...] = (m2 / ssum).reshape(TBLK)

    row128 = lax.broadcasted_iota(jnp.int32, (128, 128), 0)
    col128 = lax.broadcasted_iota(jnp.int32, (128, 128), 1)
    triu_incl = (row128 <= col128).astype(jnp.float32)
    for sel, carry, rref in ((i1, carry0, rank0_ref), (i2, carry1, rank1_ref)):
        oh = (row128 == sel).astype(jnp.float32)
        cum = jnp.dot(oh, triu_incl, preferred_element_type=jnp.float32)
        cb = carry[...]
        rank = jnp.sum(oh * (cum + cb - 1.0), axis=0, keepdims=True)
        rref[...] = rank.astype(jnp.int32).reshape(TBLK)
        carry[...] = cb + jnp.sum(oh, axis=1, keepdims=True)

    @pl.when(i == NTB - 1)
    def _():
        c0c = carry0[...]
        tot = (c0c + carry1[...]).astype(jnp.int32)
        padded = ((tot + BS - 1) >> 8) << 8
        nblk = (padded >> 8).astype(jnp.float32)
        low_strict = (row128 > col128).astype(jnp.float32)
        offs_col = jnp.dot(low_strict, padded.astype(jnp.float32),
                           preferred_element_type=jnp.float32)
        eye = (row128 == col128).astype(jnp.float32)
        c0_ref[...] = jnp.sum(eye * c0c, axis=0).astype(jnp.int32)
        offs_ref[...] = jnp.sum(eye * offs_col, axis=0).astype(jnp.int32)
        nblk_ref[...] = jnp.sum(eye * nblk, axis=0).astype(jnp.int32)


def _router(x2d, router_w):
    flat_i = jax.ShapeDtypeStruct((T,), jnp.int32)
    flat_f = jax.ShapeDtypeStruct((T,), jnp.float32)
    meta_i = jax.ShapeDtypeStruct((128,), jnp.int32)
    blk = pl.BlockSpec((TBLK,), lambda i: (i,))
    meta = pl.BlockSpec((128,), lambda i: (0,))
    return pl.pallas_call(
        _router_body,
        grid=(NTB,),
        in_specs=[
            pl.BlockSpec((TBLK, D), lambda i: (i, 0)),
            pl.BlockSpec((E, D), lambda i: (0, 0)),
        ],
        out_specs=[blk, blk, blk, blk, blk, blk, meta, meta, meta],
        out_shape=[flat_i, flat_i, flat_i, flat_i, flat_f, flat_f,
                   meta_i, meta_i, meta_i],
        scratch_shapes=[
            pltpu.VMEM((128, 1), jnp.float32),
            pltpu.VMEM((128, 1), jnp.float32),
        ],
    )(x2d, router_w)


def _dispatch_body(sel0_h, sel1_h, rank0_h, rank1_h, c0_h, offs_h, x_h,
                   xs_h, dst_h, selc_v, rankc_v, c0_v, offs_v,
                   dst0_v, dst1_v, tok0_v, tok1_v, rows0_v, rows1_v,
                   semg0, semg1, sems0, sems1):
    wid = lax.axis_index("s") * 2 + lax.axis_index("c")
    kflag = wid // 16
    tb = (wid % 16) * 128
    pb = wid * 128

    @pl.when(kflag == 0)
    def _():
        pltpu.sync_copy(sel0_h.at[pl.ds(tb, 128)], selc_v)
        pltpu.sync_copy(rank0_h.at[pl.ds(tb, 128)], rankc_v)

    @pl.when(kflag == 1)
    def _():
        pltpu.sync_copy(sel1_h.at[pl.ds(tb, 128)], selc_v)
        pltpu.sync_copy(rank1_h.at[pl.ds(tb, 128)], rankc_v)

    pltpu.sync_copy(c0_h.at[pl.ds(0, 16)], c0_v)
    pltpu.sync_copy(offs_h.at[pl.ds(0, 16)], offs_v)
    kvec = jnp.full((16,), kflag, dtype=jnp.int32)
    for j in range(8):
        dvh, tvh = (dst0_v, tok0_v) if j < 4 else (dst1_v, tok1_v)
        jh = j % 4
        s16 = selc_v[pl.ds(j * 16, 16)]
        r16 = rankc_v[pl.ds(j * 16, 16)]
        o16 = plsc.load_gather(offs_v, [s16])
        c016 = plsc.load_gather(c0_v, [s16])
        dvh[pl.ds(jh * 16, 16)] = o16 + c016 * kvec + r16
        tvh[pl.ds(jh * 16, 16)] = tb + j * 16 + lax.iota(jnp.int32, 16)
    g0 = pltpu.async_copy(x_h.at[tok0_v], rows0_v, semg0)
    g1 = pltpu.async_copy(x_h.at[tok1_v], rows1_v, semg1)
    g0.wait()
    s0 = pltpu.async_copy(rows0_v, xs_h.at[dst0_v], sems0)
    g1.wait()
    s1 = pltpu.async_copy(rows1_v, xs_h.at[dst1_v], sems1)
    pltpu.sync_copy(dst0_v, dst_h.at[pl.ds(pb, 64)])
    pltpu.sync_copy(dst1_v, dst_h.at[pl.ds(pb + 64, 64)])
    s0.wait()
    s1.wait()


def _dispatch(sel0, sel1, rank0, rank1, c0v, offsv, x2d):
    f = pl.kernel(
        _dispatch_body,
        out_type=[
            jax.ShapeDtypeStruct((P, D), jnp.float32),
            jax.ShapeDtypeStruct((NPAIR,), jnp.int32),
        ],
        mesh=plsc.VectorSubcoreMesh(core_axis_name="c", subcore_axis_name="s"),
        compiler_params=pltpu.CompilerParams(needs_layout_passes=False),
        scratch_types=[
            pltpu.VMEM((128,), jnp.int32),
            pltpu.VMEM((128,), jnp.int32),
            pltpu.VMEM((16,), jnp.int32),
            pltpu.VMEM((16,), jnp.int32),
            pltpu.VMEM((64,), jnp.int32),
            pltpu.VMEM((64,), jnp.int32),
            pltpu.VMEM((64,), jnp.int32),
            pltpu.VMEM((64,), jnp.int32),
            pltpu.VMEM((64, D), jnp.float32),
            pltpu.VMEM((64, D), jnp.float32),
            pltpu.SemaphoreType.DMA,
            pltpu.SemaphoreType.DMA,
            pltpu.SemaphoreType.DMA,
            pltpu.SemaphoreType.DMA,
        ],
    )
    return f(sel0, sel1, rank0, rank1, c0v, offsv, x2d)


def _ffn_body(nb_ref, xs_hbm, w1_hbm, w2_hbm, y_hbm, w1b, w2b, xsb, yb,
              w1s, w2s, xss, yss):
    def w1cp(e, slot):
        return pltpu.make_async_copy(
            w1_hbm.at[:, pl.ds(e * DFFN, DFFN)], w1b.at[slot], w1s.at[slot])

    def w2cp(e, slot):
        return pltpu.make_async_copy(
            w2_hbm.at[pl.ds(e * DFFN, DFFN), :], w2b.at[slot], w2s.at[slot])

    def xscp(g, slot):
        return pltpu.make_async_copy(
            xs_hbm.at[pl.ds(g * BS, BS)], xsb.at[slot], xss.at[slot])

    def ycp(g, slot):
        return pltpu.make_async_copy(
            yb.at[slot], y_hbm.at[pl.ds(g * BS, BS)], yss.at[slot])

    nbtot = nb_ref[0]
    for e in range(1, E):
        nbtot = nbtot + nb_ref[e]

    for e in range(3):
        w1cp(e, e).start()
        w2cp(e, e).start()
    xscp(0, 0).start()

    g = 0
    for e in range(E):
        slot = e % 3
        w1cp(e, slot).wait()
        w2cp(e, slot).wait()

        def blk(j, g, slot=slot):
            xslot = lax.rem(g, 2)
            xscp(g, xslot).wait()

            @pl.when(g + 1 < nbtot)
            def _():
                xscp(g + 1, lax.rem(g + 1, 2)).start()

            @pl.when(g >= 2)
            def _():
                ycp(g - 2, xslot).wait()

            h = jnp.dot(xsb[xslot].astype(jnp.bfloat16),
                        w1b[slot].astype(jnp.bfloat16),
                        preferred_element_type=jnp.float32)
            h = jax.nn.gelu(h)
            yb[xslot] = jnp.dot(h.astype(jnp.bfloat16),
                                w2b[slot].astype(jnp.bfloat16),
                                preferred_element_type=jnp.float32)
            ycp(g, xslot).start()
            return g + 1

        g = lax.fori_loop(0, nb_ref[e], blk, g)
        if e + 3 < E:
            w1cp(e + 3, slot).start()
            w2cp(e + 3, slot).start()

    ycp(g - 1, lax.rem(g - 1, 2)).wait()
    ycp(g - 2, lax.rem(g - 2, 2)).wait()


def _ffn(nblk, xs, w1, w2):
    return pl.pallas_call(
        _ffn_body,
        in_specs=[
            pl.BlockSpec(memory_space=pltpu.SMEM),
            pl.BlockSpec(memory_space=pl.ANY),
            pl.BlockSpec(memory_space=pl.ANY),
            pl.BlockSpec(memory_space=pl.ANY),
        ],
        out_specs=pl.BlockSpec(memory_space=pl.ANY),
        out_shape=jax.ShapeDtypeStruct((P, D), jnp.float32),
        scratch_shapes=[
            pltpu.VMEM((3, D, DFFN), jnp.float32),
            pltpu.VMEM((3, DFFN, D), jnp.float32),
            pltpu.VMEM((2, BS, D), jnp.float32),
            pltpu.VMEM((2, BS, D), jnp.float32),
            pltpu.SemaphoreType.DMA((3,)),
            pltpu.SemaphoreType.DMA((3,)),
            pltpu.SemaphoreType.DMA((2,)),
            pltpu.SemaphoreType.DMA((2,)),
        ],
    )(nblk, xs, w1, w2)


def _combine_body(y_h, dst_h, rw0_h, rw1_h, o_h, i0a_v, i1a_v, i0b_v, i1b_v,
                  w0_v, w1_v, r0a_v, r1a_v, r0b_v, r1b_v,
                  sga0, sga1, sgb0, sgb1, soa, sob):
    wid = lax.axis_index("s") * 2 + lax.axis_index("c")
    tb = wid * 64
    pltpu.sync_copy(dst_h.at[pl.ds(tb, 32)], i0a_v)
    pltpu.sync_copy(dst_h.at[pl.ds(T + tb, 32)], i1a_v)
    pltpu.sync_copy(dst_h.at[pl.ds(tb + 32, 32)], i0b_v)
    pltpu.sync_copy(dst_h.at[pl.ds(T + tb + 32, 32)], i1b_v)
    pltpu.sync_copy(rw0_h.at[pl.ds(tb, 64)], w0_v)
    pltpu.sync_copy(rw1_h.at[pl.ds(tb, 64)], w1_v)
    ga0 = pltpu.async_copy(y_h.at[i0a_v], r0a_v, sga0)
    ga1 = pltpu.async_copy(y_h.at[i1a_v], r1a_v, sga1)
    gb0 = pltpu.async_copy(y_h.at[i0b_v], r0b_v, sgb0)
    gb1 = pltpu.async_copy(y_h.at[i1b_v], r1b_v, sgb1)

    def combine(r0_v, r1_v, woff):
        def body(j, carry):
            jw = jnp.full((16,), j + woff, dtype=jnp.int32)
            w0 = plsc.load_gather(w0_v, [jw])
            w1s = plsc.load_gather(w1_v, [jw])
            for c in range(D // 16):
                sl = pl.ds(c * 16, 16)
                r0_v[j, sl] = r0_v[j, sl] * w0 + r1_v[j, sl] * w1s
            return carry
        lax.fori_loop(0, 32, body, 0)

    ga0.wait()
    ga1.wait()
    combine(r0a_v, r1a_v, 0)
    oa = pltpu.async_copy(r0a_v, o_h.at[pl.ds(tb, 32)], soa)
    gb0.wait()
    gb1.wait()
    combine(r0b_v, r1b_v, 32)
    ob = pltpu.async_copy(r0b_v, o_h.at[pl.ds(tb + 32, 32)], sob)
    oa.wait()
    ob.wait()


def _combine(y, dst, rw0f, rw1f):
    f = pl.kernel(
        _combine_body,
        out_type=jax.ShapeDtypeStruct((T, D), jnp.float32),
        mesh=plsc.VectorSubcoreMesh(core_axis_name="c", subcore_axis_name="s"),
        compiler_params=pltpu.CompilerParams(needs_layout_passes=False),
        scratch_types=[
            pltpu.VMEM((32,), jnp.int32),
            pltpu.VMEM((32,), jnp.int32),
            pltpu.VMEM((32,), jnp.int32),
            pltpu.VMEM((32,), jnp.int32),
            pltpu.VMEM((64,), jnp.float32),
            pltpu.VMEM((64,), jnp.float32),
            pltpu.VMEM((32, D), jnp.float32),
            pltpu.VMEM((32, D), jnp.float32),
            pltpu.VMEM((32, D), jnp.float32),
            pltpu.VMEM((32, D), jnp.float32),
            pltpu.SemaphoreType.DMA,
            pltpu.SemaphoreType.DMA,
            pltpu.SemaphoreType.DMA,
            pltpu.SemaphoreType.DMA,
            pltpu.SemaphoreType.DMA,
            pltpu.SemaphoreType.DMA,
        ],
    )
    return f(y, dst, rw0f, rw1f)


def kernel(x, router_w, w1, w2):
    b, s, d = x.shape
    x2d = x.reshape(T, D)
    (sel0, sel1, rank0, rank1, rw0f, rw1f,
     c0v, offsv, nblkv) = _router(x2d, router_w)
    xs, dst = _dispatch(sel0, sel1, rank0, rank1, c0v, offsv, x2d)
    y = _ffn(nblkv, xs, w1, w2)
    out = _combine(y, dst, rw0f, rw1f)
    return out.reshape(b, s, d)


# speedup vs baseline: 1.2224x; 1.0014x over previous
import jax
import jax.numpy as jnp
from jax import lax
from jax.experimental import pallas as pl
from jax.experimental.pallas import tpu as pltpu
from jax.experimental.pallas import tpu_sc as plsc

E = 8
K = 2
D = 768
BS = 256
DFFN = 1536
T = 2048
NPAIR = T * K
NB = 23
P = NB * BS
TBLK = 128
NTB = T // TBLK
NW = 32


def _router_body(x_ref, rwin_ref, sel0_ref, sel1_ref, rank0_ref, rank1_ref,
                 rw0_ref, rw1_ref, c0_ref, offs_ref, nblk_ref,
                 carry0, carry1):
    i = pl.program_id(0)

    @pl.when(i == 0)
    def _():
        carry0[...] = jnp.zeros_like(carry0)
        carry1[...] = jnp.zeros_like(carry1)

    xb = x_ref[...]
    logits = lax.dot_general(rwin_ref[...], xb, (((1,), (1,)), ((), ())),
                             preferred_element_type=jnp.float32)
    m = jnp.max(logits, axis=0, keepdims=True)
    ex = jnp.exp(logits - m)
    p = ex / jnp.sum(ex, axis=0, keepdims=True)
    sub8 = lax.broadcasted_iota(jnp.int32, (E, TBLK), 0)
    m1 = jnp.max(p, axis=0, keepdims=True)
    i1 = jnp.min(jnp.where(p >= m1, sub8, E), axis=0, keepdims=True)
    p2 = jnp.where(sub8 == i1, -1.0, p)
    m2 = jnp.max(p2, axis=0, keepdims=True)
    i2 = jnp.min(jnp.where(p2 >= m2, sub8, E), axis=0, keepdims=True)
    ssum = m1 + m2
    sel0_ref[...] = i1.reshape(TBLK)
    sel1_ref[...] = i2.reshape(TBLK)
    rw0_ref[...] = (m1 / ssum).reshape(TBLK)
    rw1_ref[...] = (m2 / ssum).reshape(TBLK)

    row128 = lax.broadcasted_iota(jnp.int32, (128, 128), 0)
    col128 = lax.broadcasted_iota(jnp.int32, (128, 128), 1)
    triu_incl = (row128 <= col128).astype(jnp.float32)
    for sel, carry, rref in ((i1, carry0, rank0_ref), (i2, carry1, rank1_ref)):
        oh = (row128 == sel).astype(jnp.float32)
        cum = jnp.dot(oh, triu_incl, preferred_element_type=jnp.float32)
        cb = carry[...]
        rank = jnp.sum(oh * (cum + cb - 1.0), axis=0, keepdims=True)
        rref[...] = rank.astype(jnp.int32).reshape(TBLK)
        carry[...] = cb + jnp.sum(oh, axis=1, keepdims=True)

    @pl.when(i == NTB - 1)
    def _():
        c0c = carry0[...]
        tot = (c0c + carry1[...]).astype(jnp.int32)
        padded = ((tot + BS - 1) >> 8) << 8
        nblk = (padded >> 8).astype(jnp.float32)
        low_strict = (row128 > col128).astype(jnp.float32)
        offs_col = jnp.dot(low_strict, padded.astype(jnp.float32),
                           preferred_element_type=jnp.float32)
        eye = (row128 == col128).astype(jnp.float32)
        c0_ref[...] = jnp.sum(eye * c0c, axis=0).astype(jnp.int32)
        offs_ref[...] = jnp.sum(eye * offs_col, axis=0).astype(jnp.int32)
        nblk_ref[...] = jnp.sum(eye * nblk, axis=0).astype(jnp.int32)


def _router(x2d, router_w):
    flat_i = jax.ShapeDtypeStruct((T,), jnp.int32)
    flat_f = jax.ShapeDtypeStruct((T,), jnp.float32)
    meta_i = jax.ShapeDtypeStruct((128,), jnp.int32)
    blk = pl.BlockSpec((TBLK,), lambda i: (i,))
    meta = pl.BlockSpec((128,), lambda i: (0,))
    return pl.pallas_call(
        _router_body,
        grid=(NTB,),
        in_specs=[
            pl.BlockSpec((TBLK, D), lambda i: (i, 0)),
            pl.BlockSpec((E, D), lambda i: (0, 0)),
        ],
        out_specs=[blk, blk, blk, blk, blk, blk, meta, meta, meta],
        out_shape=[flat_i, flat_i, flat_i, flat_i, flat_f, flat_f,
                   meta_i, meta_i, meta_i],
        scratch_shapes=[
            pltpu.VMEM((128, 1), jnp.float32),
            pltpu.VMEM((128, 1), jnp.float32),
        ],
    )(x2d, router_w)


def _dispatch_body(sel0_h, sel1_h, rank0_h, rank1_h, c0_h, offs_h, x_h,
                   xs_h, dst_h, selc_v, rankc_v, c0_v, offs_v,
                   dst0_v, dst1_v, tok0_v, tok1_v, rows0_v, rows1_v,
                   semg0, semg1, sems0, sems1):
    wid = lax.axis_index("s") * 2 + lax.axis_index("c")
    kflag = wid // 16
    tb = (wid % 16) * 128
    pb = wid * 128

    @pl.when(kflag == 0)
    def _():
        pltpu.sync_copy(sel0_h.at[pl.ds(tb, 128)], selc_v)
        pltpu.sync_copy(rank0_h.at[pl.ds(tb, 128)], rankc_v)

    @pl.when(kflag == 1)
    def _():
        pltpu.sync_copy(sel1_h.at[pl.ds(tb, 128)], selc_v)
        pltpu.sync_copy(rank1_h.at[pl.ds(tb, 128)], rankc_v)

    pltpu.sync_copy(c0_h.at[pl.ds(0, 16)], c0_v)
    pltpu.sync_copy(offs_h.at[pl.ds(0, 16)], offs_v)
    kvec = jnp.full((16,), kflag, dtype=jnp.int32)
    for j in range(8):
        dvh, tvh = (dst0_v, tok0_v) if j < 4 else (dst1_v, tok1_v)
        jh = j % 4
        s16 = selc_v[pl.ds(j * 16, 16)]
        r16 = rankc_v[pl.ds(j * 16, 16)]
        o16 = plsc.load_gather(offs_v, [s16])
        c016 = plsc.load_gather(c0_v, [s16])
        dvh[pl.ds(jh * 16, 16)] = o16 + c016 * kvec + r16
        tvh[pl.ds(jh * 16, 16)] = tb + j * 16 + lax.iota(jnp.int32, 16)
    g0 = pltpu.async_copy(x_h.at[tok0_v], rows0_v, semg0)
    g1 = pltpu.async_copy(x_h.at[tok1_v], rows1_v, semg1)
    g0.wait()
    s0 = pltpu.async_copy(rows0_v, xs_h.at[dst0_v], sems0)
    g1.wait()
    s1 = pltpu.async_copy(rows1_v, xs_h.at[dst1_v], sems1)
    pltpu.sync_copy(dst0_v, dst_h.at[pl.ds(pb, 64)])
    pltpu.sync_copy(dst1_v, dst_h.at[pl.ds(pb + 64, 64)])
    s0.wait()
    s1.wait()


def _dispatch(sel0, sel1, rank0, rank1, c0v, offsv, x2d):
    f = pl.kernel(
        _dispatch_body,
        out_type=[
            jax.ShapeDtypeStruct((P, D), jnp.float32),
            jax.ShapeDtypeStruct((NPAIR,), jnp.int32),
        ],
        mesh=plsc.VectorSubcoreMesh(core_axis_name="c", subcore_axis_name="s"),
        compiler_params=pltpu.CompilerParams(needs_layout_passes=False),
        scratch_types=[
            pltpu.VMEM((128,), jnp.int32),
            pltpu.VMEM((128,), jnp.int32),
            pltpu.VMEM((16,), jnp.int32),
            pltpu.VMEM((16,), jnp.int32),
            pltpu.VMEM((64,), jnp.int32),
            pltpu.VMEM((64,), jnp.int32),
            pltpu.VMEM((64,), jnp.int32),
            pltpu.VMEM((64,), jnp.int32),
            pltpu.VMEM((64, D), jnp.float32),
            pltpu.VMEM((64, D), jnp.float32),
            pltpu.SemaphoreType.DMA,
            pltpu.SemaphoreType.DMA,
            pltpu.SemaphoreType.DMA,
            pltpu.SemaphoreType.DMA,
        ],
    )
    return f(sel0, sel1, rank0, rank1, c0v, offsv, x2d)


def _ffn_body(nb_ref, xs_hbm, w1_hbm, w2_hbm, y_hbm, w1b, w2b, xsb, yb,
              w1sa, w1sb, w2sa, w2sb, xss, yss):
    H = DFFN // 2

    class _Pair:
        def __init__(self, a, b):
            self.a, self.b = a, b

        def start(self):
            self.a.start()
            self.b.start()

        def wait(self):
            self.a.wait()
            self.b.wait()

    def w1cp(e, slot):
        return _Pair(
            pltpu.make_async_copy(w1_hbm.at[:, pl.ds(e * DFFN, H)],
                                  w1b.at[slot, :, pl.ds(0, H)],
                                  w1sa.at[slot]),
            pltpu.make_async_copy(w1_hbm.at[:, pl.ds(e * DFFN + H, H)],
                                  w1b.at[slot, :, pl.ds(H, H)],
                                  w1sb.at[slot]))

    def w2cp(e, slot):
        return _Pair(
            pltpu.make_async_copy(w2_hbm.at[pl.ds(e * DFFN, H), :],
                                  w2b.at[slot, pl.ds(0, H)],
                                  w2sa.at[slot]),
            pltpu.make_async_copy(w2_hbm.at[pl.ds(e * DFFN + H, H), :],
                                  w2b.at[slot, pl.ds(H, H)],
                                  w2sb.at[slot]))

    def xscp(g, slot):
        return pltpu.make_async_copy(
            xs_hbm.at[pl.ds(g * BS, BS)], xsb.at[slot], xss.at[slot])

    def ycp(g, slot):
        return pltpu.make_async_copy(
            yb.at[slot], y_hbm.at[pl.ds(g * BS, BS)], yss.at[slot])

    nbtot = nb_ref[0]
    for e in range(1, E):
        nbtot = nbtot + nb_ref[e]

    for e in range(3):
        w1cp(e, e).start()
        w2cp(e, e).start()
    xscp(0, 0).start()

    g = 0
    for e in range(E):
        slot = e % 3
        w1cp(e, slot).wait()
        w2cp(e, slot).wait()

        def blk(j, g, slot=slot):
            xslot = lax.rem(g, 2)
            xscp(g, xslot).wait()

            @pl.when(g + 1 < nbtot)
            def _():
                xscp(g + 1, lax.rem(g + 1, 2)).start()

            @pl.when(g >= 2)
            def _():
                ycp(g - 2, xslot).wait()

            h = jnp.dot(xsb[xslot].astype(jnp.bfloat16),
                        w1b[slot].astype(jnp.bfloat16),
                        preferred_element_type=jnp.float32)
            h = jax.nn.gelu(h)
            yb[xslot] = jnp.dot(h.astype(jnp.bfloat16),
                                w2b[slot].astype(jnp.bfloat16),
                                preferred_element_type=jnp.float32)
            ycp(g, xslot).start()
            return g + 1

        g = lax.fori_loop(0, nb_ref[e], blk, g)
        if e + 3 < E:
            w1cp(e + 3, slot).start()
            w2cp(e + 3, slot).start()

    ycp(g - 1, lax.rem(g - 1, 2)).wait()
    ycp(g - 2, lax.rem(g - 2, 2)).wait()


def _ffn(nblk, xs, w1, w2):
    return pl.pallas_call(
        _ffn_body,
        in_specs=[
            pl.BlockSpec(memory_space=pltpu.SMEM),
            pl.BlockSpec(memory_space=pl.ANY),
            pl.BlockSpec(memory_space=pl.ANY),
            pl.BlockSpec(memory_space=pl.ANY),
        ],
        out_specs=pl.BlockSpec(memory_space=pl.ANY),
        out_shape=jax.ShapeDtypeStruct((P, D), jnp.float32),
        scratch_shapes=[
            pltpu.VMEM((3, D, DFFN), jnp.float32),
            pltpu.VMEM((3, DFFN, D), jnp.float32),
            pltpu.VMEM((2, BS, D), jnp.float32),
            pltpu.VMEM((2, BS, D), jnp.float32),
            pltpu.SemaphoreType.DMA((3,)),
            pltpu.SemaphoreType.DMA((3,)),
            pltpu.SemaphoreType.DMA((3,)),
            pltpu.SemaphoreType.DMA((3,)),
            pltpu.SemaphoreType.DMA((2,)),
            pltpu.SemaphoreType.DMA((2,)),
        ],
    )(nblk, xs, w1, w2)


def _combine_body(y_h, dst_h, rw0_h, rw1_h, o_h, i0a_v, i1a_v, i0b_v, i1b_v,
                  w0_v, w1_v, r0a_v, r1a_v, r0b_v, r1b_v,
                  sga0, sga1, sgb0, sgb1, soa, sob):
    wid = lax.axis_index("s") * 2 + lax.axis_index("c")
    tb = wid * 64
    pltpu.sync_copy(dst_h.at[pl.ds(tb, 32)], i0a_v)
    pltpu.sync_copy(dst_h.at[pl.ds(T + tb, 32)], i1a_v)
    pltpu.sync_copy(dst_h.at[pl.ds(tb + 32, 32)], i0b_v)
    pltpu.sync_copy(dst_h.at[pl.ds(T + tb + 32, 32)], i1b_v)
    pltpu.sync_copy(rw0_h.at[pl.ds(tb, 64)], w0_v)
    pltpu.sync_copy(rw1_h.at[pl.ds(tb, 64)], w1_v)
    ga0 = pltpu.async_copy(y_h.at[i0a_v], r0a_v, sga0)
    ga1 = pltpu.async_copy(y_h.at[i1a_v], r1a_v, sga1)
    gb0 = pltpu.async_copy(y_h.at[i0b_v], r0b_v, sgb0)
    gb1 = pltpu.async_copy(y_h.at[i1b_v], r1b_v, sgb1)

    def combine(r0_v, r1_v, woff):
        def body(j, carry):
            jw = jnp.full((16,), j + woff, dtype=jnp.int32)
            w0 = plsc.load_gather(w0_v, [jw])
            w1s = plsc.load_gather(w1_v, [jw])
            for c in range(D // 16):
                sl = pl.ds(c * 16, 16)
                r0_v[j, sl] = r0_v[j, sl] * w0 + r1_v[j, sl] * w1s
            return carry
        lax.fori_loop(0, 32, body, 0)

    ga0.wait()
    ga1.wait()
    combine(r0a_v, r1a_v, 0)
    oa = pltpu.async_copy(r0a_v, o_h.at[pl.ds(tb, 32)], soa)
    gb0.wait()
    gb1.wait()
    combine(r0b_v, r1b_v, 32)
    ob = pltpu.async_copy(r0b_v, o_h.at[pl.ds(tb + 32, 32)], sob)
    oa.wait()
    ob.wait()


def _combine(y, dst, rw0f, rw1f):
    f = pl.kernel(
        _combine_body,
        out_type=jax.ShapeDtypeStruct((T, D), jnp.float32),
        mesh=plsc.VectorSubcoreMesh(core_axis_name="c", subcore_axis_name="s"),
        compiler_params=pltpu.CompilerParams(needs_layout_passes=False),
        scratch_types=[
            pltpu.VMEM((32,), jnp.int32),
            pltpu.VMEM((32,), jnp.int32),
            pltpu.VMEM((32,), jnp.int32),
            pltpu.VMEM((32,), jnp.int32),
            pltpu.VMEM((64,), jnp.float32),
            pltpu.VMEM((64,), jnp.float32),
            pltpu.VMEM((32, D), jnp.float32),
            pltpu.VMEM((32, D), jnp.float32),
            pltpu.VMEM((32, D), jnp.float32),
            pltpu.VMEM((32, D), jnp.float32),
            pltpu.SemaphoreType.DMA,
            pltpu.SemaphoreType.DMA,
            pltpu.SemaphoreType.DMA,
            pltpu.SemaphoreType.DMA,
            pltpu.SemaphoreType.DMA,
            pltpu.SemaphoreType.DMA,
        ],
    )
    return f(y, dst, rw0f, rw1f)


def kernel(x, router_w, w1, w2):
    b, s, d = x.shape
    x2d = x.reshape(T, D)
    (sel0, sel1, rank0, rank1, rw0f, rw1f,
     c0v, offsv, nblkv) = _router(x2d, router_w)
    xs, dst = _dispatch(sel0, sel1, rank0, rank1, c0v, offsv, x2d)
    y = _ffn(nblkv, xs, w1, w2)
    out = _combine(y, dst, rw0f, rw1f)
    return out.reshape(b, s, d)
